# baseline (speedup 1.0000x reference)
import jax
import jax.numpy as jnp
from jax.experimental import pallas as pl


_TILE_T = 256


def _add_pe_kernel(x_ref, w_ref, out_ref):
    w = w_ref[...]
    s0 = x_ref[0] + w
    s1 = x_ref[1] + w
    out_ref[0, 0] = s0
    out_ref[0, 1] = s1
    out_ref[1, 0] = s0
    out_ref[1, 1] = s1


def kernel(x, embed_weight):
    B, T, E = x.shape
    tile_t = min(_TILE_T, T)
    grid = (T // tile_t,)
    return pl.pallas_call(
        _add_pe_kernel,
        grid=grid,
        in_specs=[
            pl.BlockSpec((B, tile_t, E), lambda t: (0, t, 0)),
            pl.BlockSpec((tile_t, E), lambda t: (t, 0)),
        ],
        out_specs=pl.BlockSpec((B, B, tile_t, E), lambda t: (0, 0, t, 0)),
        out_shape=jax.ShapeDtypeStruct((B, B, T, E), x.dtype),
    )(x, embed_weight)


# speedup vs baseline: 1.1668x; 1.1668x over previous
import jax
import jax.numpy as jnp
from jax.experimental import pallas as pl


_TILE_T = 1024


def _add_pe_kernel(x_ref, w_ref, out_ref):
    w = w_ref[...]
    s0 = x_ref[0] + w
    s1 = x_ref[1] + w
    out_ref[0, 0] = s0
    out_ref[0, 1] = s1
    out_ref[1, 0] = s0
    out_ref[1, 1] = s1


def kernel(x, embed_weight):
    B, T, E = x.shape
    tile_t = min(_TILE_T, T)
    grid = (T // tile_t,)
    return pl.pallas_call(
        _add_pe_kernel,
        grid=grid,
        in_specs=[
            pl.BlockSpec((B, tile_t, E), lambda t: (0, t, 0)),
            pl.BlockSpec((tile_t, E), lambda t: (t, 0)),
        ],
        out_specs=pl.BlockSpec((B, B, tile_t, E), lambda t: (0, 0, t, 0)),
        out_shape=jax.ShapeDtypeStruct((B, B, T, E), x.dtype),
    )(x, embed_weight)
